# baseline (speedup 1.0000x reference)
import functools

import jax
import jax.numpy as jnp
import numpy as np
from jax import lax
from jax.experimental import pallas as pl
from jax.experimental.pallas import tpu as pltpu
from jax.experimental.pallas import tpu_sc as plsc

_SEED = 42

_NC = 2
_NS = 16
_NW = _NC * _NS
_L = 16

_ROWS = 64
_SPLIT = 2
_RH = _ROWS // _SPLIT
_N = 100000
_K = 4000
_NCHUNK = _N // _K
_GROUPS = _K // _L
_UNROLL = 10


_U32 = np.uint32


def _threefry2x32(k1, k2, x0, x1):
    rot = ((13, 15, 26, 6), (17, 29, 16, 24))
    ks = [_U32(k1), _U32(k2), _U32(k1) ^ _U32(k2) ^ _U32(0x1BD11BDA)]
    x = [x0.astype(np.uint32) + ks[0], x1.astype(np.uint32) + ks[1]]
    rots = [rot[0], rot[1]]
    ksr = [ks[1], ks[2], ks[0]]
    for i in range(5):
        for r in rots[0]:
            x[0] = x[0] + x[1]
            x[1] = (x[1] << _U32(r)) | (x[1] >> _U32(32 - r))
            x[1] = x[0] ^ x[1]
        x = [x[0] + ksr[0], x[1] + ksr[1] + _U32(i + 1)]
        ksr = ksr[1:] + ksr[:1]
        rots = rots[1:] + rots[:1]
    return x[0], x[1]


def _perm_host(seed: int, n: int) -> np.ndarray:
    key = (_U32(0), _U32(seed))
    x = np.arange(n, dtype=np.int32)
    num_rounds = int(np.ceil(3 * np.log(max(1, n)) / np.log(np.iinfo(np.uint32).max)))
    hi2, lo2 = np.zeros(2, np.uint32), np.arange(2, dtype=np.uint32)
    hi_n, lo_n = np.zeros(n, np.uint32), np.arange(n, dtype=np.uint32)
    for _ in range(num_rounds):
        b1, b2 = _threefry2x32(key[0], key[1], hi2, lo2)
        key, subkey = (b1[0], b2[0]), (b1[1], b2[1])
        s1, s2 = _threefry2x32(subkey[0], subkey[1], hi_n, lo_n)
        x = x[np.argsort(s1 ^ s2, kind="stable")]
    return x


_PERM = _perm_host(_SEED, _N)


def _shuffle_body(pc_hbm, perm_hbm, out_hbm, row_v,
                  idx0_v, idx1_v, gout0_v, gout1_v, sp_perm,
                  row_sem, isem0, isem1, osem0, osem1):
    idx_v = (idx0_v, idx1_v)
    gout_v = (gout0_v, gout1_v)
    isem = (isem0, isem1)
    osem = (osem0, osem1)
    wid = lax.axis_index("s") * _NC + lax.axis_index("c")

    @pl.when(lax.axis_index("s") == 0)
    def _stage_perm():
        pltpu.sync_copy(perm_hbm, sp_perm)

    plsc.subcore_barrier()

    def idx_copy(c, b):
        return pltpu.make_async_copy(
            sp_perm.at[pl.ds(c * _K, _K)], idx_v[b], isem[b])

    def out_copy(r, c, b):
        return pltpu.make_async_copy(
            gout_v[b], out_hbm.at[pl.ds(r * _N + c * _K, _K)], osem[b])

    nstores = 0
    for t in range(_RH // _NW):
        r = wid * (_RH // _NW) + t
        rowcp = pltpu.make_async_copy(
            pc_hbm.at[pl.ds(r * _N, _N)], row_v, row_sem)
        rowcp.start()
        idx_copy(0, 0).start()
        rowcp.wait()
        for c in range(_NCHUNK):
            b = c % 2
            idx_copy(c, b).wait()
            if c + 1 < _NCHUNK:
                idx_copy(c + 1, (c + 1) % 2).start()
            if nstores >= 2:
                out_copy(r, c, b).wait()

            @plsc.parallel_loop(0, _K, step=_L, unroll=_UNROLL)
            def _gather_loop(off, _b=b):
                idx = idx_v[_b][pl.ds(off, _L)]
                gout_v[_b][pl.ds(off, _L)] = plsc.load_gather(row_v, [idx])
            out_copy(r, c, b).start()
            nstores += 1
    out_copy(0, 0, 0).wait()
    out_copy(0, 0, 1).wait()


def kernel(pc):
    rows, n = pc.shape
    assert (rows, n) == (_ROWS, _N)
    perm = jnp.asarray(_PERM)

    shuffle = pl.kernel(
        _shuffle_body,
        out_type=jax.ShapeDtypeStruct((_RH * n,), jnp.float32),
        mesh=plsc.VectorSubcoreMesh(
            core_axis_name="c", subcore_axis_name="s",
            num_cores=_NC, num_subcores=_NS,
        ),
        scratch_types=[
            pltpu.VMEM((_N,), jnp.float32),
            pltpu.VMEM((_K,), jnp.int32),
            pltpu.VMEM((_K,), jnp.int32),
            pltpu.VMEM((_K,), jnp.float32),
            pltpu.VMEM((_K,), jnp.float32),
            pltpu.VMEM_SHARED((_N,), jnp.int32),
            pltpu.SemaphoreType.DMA,
            pltpu.SemaphoreType.DMA,
            pltpu.SemaphoreType.DMA,
            pltpu.SemaphoreType.DMA,
            pltpu.SemaphoreType.DMA,
        ],
        compiler_params=pltpu.CompilerParams(
            needs_layout_passes=False, skip_device_barrier=True),
    )
    parts = [
        shuffle(pc[i * _RH:(i + 1) * _RH].reshape(-1), perm).reshape(_RH, n)
        for i in range(_SPLIT)
    ]
    return jnp.concatenate(parts, axis=0)


# speedup vs baseline: 1.0001x; 1.0001x over previous
import functools

import jax
import jax.numpy as jnp
import numpy as np
from jax import lax
from jax.experimental import pallas as pl
from jax.experimental.pallas import tpu as pltpu
from jax.experimental.pallas import tpu_sc as plsc

_SEED = 42

_NC = 2
_NS = 16
_NW = _NC * _NS
_L = 16

_ROWS = 64
_SPLIT = 2
_RH = _ROWS // _SPLIT
_N = 100000
_K = 4000
_NCHUNK = _N // _K
_GROUPS = _K // _L
_UNROLL = 10


_U32 = np.uint32


def _threefry2x32(k1, k2, x0, x1):
    rot = ((13, 15, 26, 6), (17, 29, 16, 24))
    ks = [_U32(k1), _U32(k2), _U32(k1) ^ _U32(k2) ^ _U32(0x1BD11BDA)]
    x = [x0.astype(np.uint32) + ks[0], x1.astype(np.uint32) + ks[1]]
    rots = [rot[0], rot[1]]
    ksr = [ks[1], ks[2], ks[0]]
    for i in range(5):
        for r in rots[0]:
            x[0] = x[0] + x[1]
            x[1] = (x[1] << _U32(r)) | (x[1] >> _U32(32 - r))
            x[1] = x[0] ^ x[1]
        x = [x[0] + ksr[0], x[1] + ksr[1] + _U32(i + 1)]
        ksr = ksr[1:] + ksr[:1]
        rots = rots[1:] + rots[:1]
    return x[0], x[1]


def _perm_host(seed: int, n: int) -> np.ndarray:
    key = (_U32(0), _U32(seed))
    x = np.arange(n, dtype=np.int32)
    num_rounds = int(np.ceil(3 * np.log(max(1, n)) / np.log(np.iinfo(np.uint32).max)))
    hi2, lo2 = np.zeros(2, np.uint32), np.arange(2, dtype=np.uint32)
    hi_n, lo_n = np.zeros(n, np.uint32), np.arange(n, dtype=np.uint32)
    for _ in range(num_rounds):
        b1, b2 = _threefry2x32(key[0], key[1], hi2, lo2)
        key, subkey = (b1[0], b2[0]), (b1[1], b2[1])
        s1, s2 = _threefry2x32(subkey[0], subkey[1], hi_n, lo_n)
        x = x[np.argsort(s1 ^ s2, kind="stable")]
    return x


_PERM = _perm_host(_SEED, _N)


def _shuffle_body(pc_hbm, perm_hbm, out_hbm, row_v,
                  idx0_v, idx1_v, gout0_v, gout1_v, sp_perm,
                  row_sem, isem0, isem1, osem0, osem1):
    idx_v = (idx0_v, idx1_v)
    gout_v = (gout0_v, gout1_v)
    isem = (isem0, isem1)
    osem = (osem0, osem1)
    wid = lax.axis_index("s") * _NC + lax.axis_index("c")

    @pl.when(lax.axis_index("s") == 0)
    def _stage_perm():
        pltpu.sync_copy(perm_hbm, sp_perm)

    plsc.subcore_barrier()

    def idx_copy(c, b):
        return pltpu.make_async_copy(
            sp_perm.at[pl.ds(c * _K, _K)], idx_v[b], isem[b])

    def out_copy(r, c, b):
        return pltpu.make_async_copy(
            gout_v[b], out_hbm.at[pl.ds(r * _N + c * _K, _K)], osem[b])

    nstores = 0
    for t in range(_RH // _NW):
        r = wid * (_RH // _NW) + t
        rowcp = pltpu.make_async_copy(
            pc_hbm.at[pl.ds(r * _N, _N)], row_v, row_sem)
        rowcp.start()
        idx_copy(0, 0).start()
        rowcp.wait()
        for c in range(_NCHUNK):
            b = c % 2
            idx_copy(c, b).wait()
            if c + 1 < _NCHUNK:
                idx_copy(c + 1, (c + 1) % 2).start()
            if nstores >= 2:
                out_copy(r, c, b).wait()

            @plsc.parallel_loop(0, _K, step=_L, unroll=_UNROLL)
            def _gather_loop(off, _b=b):
                idx = idx_v[_b][pl.ds(off, _L)]
                gout_v[_b][pl.ds(off, _L)] = plsc.load_gather(row_v, [idx])
            out_copy(r, c, b).start()
            nstores += 1
    out_copy(0, 0, 0).wait()
    out_copy(0, 0, 1).wait()


def kernel(pc):
    rows, n = pc.shape
    assert (rows, n) == (_ROWS, _N)
    perm = jnp.asarray(_PERM)

    shuffle = pl.kernel(
        _shuffle_body,
        out_type=jax.ShapeDtypeStruct((_RH * n,), jnp.float32),
        mesh=plsc.VectorSubcoreMesh(
            core_axis_name="c", subcore_axis_name="s",
            num_cores=_NC, num_subcores=_NS,
        ),
        scratch_types=[
            pltpu.VMEM((_N,), jnp.float32),
            pltpu.VMEM((_K,), jnp.int32),
            pltpu.VMEM((_K,), jnp.int32),
            pltpu.VMEM((_K,), jnp.float32),
            pltpu.VMEM((_K,), jnp.float32),
            pltpu.VMEM_SHARED((_N,), jnp.int32),
            pltpu.SemaphoreType.DMA,
            pltpu.SemaphoreType.DMA,
            pltpu.SemaphoreType.DMA,
            pltpu.SemaphoreType.DMA,
            pltpu.SemaphoreType.DMA,
        ],
        compiler_params=pltpu.CompilerParams(
            needs_layout_passes=False, skip_device_barrier=True),
    )
    parts = [
        shuffle(pc[i * _RH:(i + 1) * _RH].reshape(-1), perm)
        for i in range(_SPLIT)
    ]
    return jnp.concatenate(parts).reshape(rows, n)


# speedup vs baseline: 1.0815x; 1.0814x over previous
import functools

import jax
import jax.numpy as jnp
import numpy as np
from jax import lax
from jax.experimental import pallas as pl
from jax.experimental.pallas import tpu as pltpu
from jax.experimental.pallas import tpu_sc as plsc

_SEED = 42

_NC = 2
_NS = 16
_NW = _NC * _NS
_L = 16

_ROWS = 64
_N = 100000
_K = 4000
_NCHUNK = _N // _K
_GROUPS = _K // _L
_UNROLL = 10


_U32 = np.uint32


def _threefry2x32(k1, k2, x0, x1):
    rot = ((13, 15, 26, 6), (17, 29, 16, 24))
    ks = [_U32(k1), _U32(k2), _U32(k1) ^ _U32(k2) ^ _U32(0x1BD11BDA)]
    x = [x0.astype(np.uint32) + ks[0], x1.astype(np.uint32) + ks[1]]
    rots = [rot[0], rot[1]]
    ksr = [ks[1], ks[2], ks[0]]
    for i in range(5):
        for r in rots[0]:
            x[0] = x[0] + x[1]
            x[1] = (x[1] << _U32(r)) | (x[1] >> _U32(32 - r))
            x[1] = x[0] ^ x[1]
        x = [x[0] + ksr[0], x[1] + ksr[1] + _U32(i + 1)]
        ksr = ksr[1:] + ksr[:1]
        rots = rots[1:] + rots[:1]
    return x[0], x[1]


def _perm_host(seed: int, n: int) -> np.ndarray:
    key = (_U32(0), _U32(seed))
    x = np.arange(n, dtype=np.int32)
    num_rounds = int(np.ceil(3 * np.log(max(1, n)) / np.log(np.iinfo(np.uint32).max)))
    hi2, lo2 = np.zeros(2, np.uint32), np.arange(2, dtype=np.uint32)
    hi_n, lo_n = np.zeros(n, np.uint32), np.arange(n, dtype=np.uint32)
    for _ in range(num_rounds):
        b1, b2 = _threefry2x32(key[0], key[1], hi2, lo2)
        key, subkey = (b1[0], b2[0]), (b1[1], b2[1])
        s1, s2 = _threefry2x32(subkey[0], subkey[1], hi_n, lo_n)
        x = x[np.argsort(s1 ^ s2, kind="stable")]
    return x


_PERM = _perm_host(_SEED, _N)


def _shuffle_body(pc_hbm, perm_hbm, out_hbm, row_v,
                  idx0_v, idx1_v, gout0_v, gout1_v, sp_perm,
                  row_sem, isem0, isem1, osem0, osem1):
    idx_v = (idx0_v, idx1_v)
    gout_v = (gout0_v, gout1_v)
    isem = (isem0, isem1)
    osem = (osem0, osem1)
    wid = lax.axis_index("s") * _NC + lax.axis_index("c")

    @pl.when(lax.axis_index("s") == 0)
    def _stage_perm():
        pltpu.sync_copy(perm_hbm, sp_perm)

    plsc.subcore_barrier()

    def idx_copy(c, b):
        return pltpu.make_async_copy(
            sp_perm.at[pl.ds(c * _K, _K)], idx_v[b], isem[b])

    def out_copy(r, c, b):
        return pltpu.make_async_copy(
            gout_v[b], out_hbm.at[pl.ds(r * _N + c * _K, _K)], osem[b])

    nstores = 0
    for t in range(_ROWS // _NW):
        r = wid * (_ROWS // _NW) + t
        rowcp = pltpu.make_async_copy(
            pc_hbm.at[pl.ds(r * _N, _N)], row_v, row_sem)
        rowcp.start()
        idx_copy(0, 0).start()
        rowcp.wait()
        for c in range(_NCHUNK):
            b = c % 2
            idx_copy(c, b).wait()
            if c + 1 < _NCHUNK:
                idx_copy(c + 1, (c + 1) % 2).start()
            if nstores >= 2:
                out_copy(r, c, b).wait()

            @plsc.parallel_loop(0, _K, step=_L, unroll=_UNROLL)
            def _gather_loop(off, _b=b):
                idx = idx_v[_b][pl.ds(off, _L)]
                gout_v[_b][pl.ds(off, _L)] = plsc.load_gather(row_v, [idx])
            out_copy(r, c, b).start()
            nstores += 1
    out_copy(0, 0, 0).wait()
    out_copy(0, 0, 1).wait()


def kernel(pc):
    rows, n = pc.shape
    assert (rows, n) == (_ROWS, _N)
    perm = jnp.asarray(_PERM)

    shuffle = pl.kernel(
        _shuffle_body,
        out_type=jax.ShapeDtypeStruct((rows * n,), jnp.float32),
        mesh=plsc.VectorSubcoreMesh(
            core_axis_name="c", subcore_axis_name="s",
            num_cores=_NC, num_subcores=_NS,
        ),
        scratch_types=[
            pltpu.VMEM((_N,), jnp.float32),
            pltpu.VMEM((_K,), jnp.int32),
            pltpu.VMEM((_K,), jnp.int32),
            pltpu.VMEM((_K,), jnp.float32),
            pltpu.VMEM((_K,), jnp.float32),
            pltpu.VMEM_SHARED((_N,), jnp.int32),
            pltpu.SemaphoreType.DMA,
            pltpu.SemaphoreType.DMA,
            pltpu.SemaphoreType.DMA,
            pltpu.SemaphoreType.DMA,
            pltpu.SemaphoreType.DMA,
        ],
        compiler_params=pltpu.CompilerParams(
            needs_layout_passes=False, skip_device_barrier=True),
    )
    return shuffle(pc.reshape(-1), perm).reshape(rows, n)
